# baseline (speedup 1.0000x reference)
import functools

import jax
import jax.numpy as jnp
from jax import lax
from jax.experimental import pallas as pl
from jax.experimental.pallas import tpu as pltpu
from jax.experimental.pallas import tpu_sc as plsc

_SUB = 128
_GROUP = 6


def _gather_pairs(table, idx_e, idx_o, n_pairs, embed):
    info = plsc.get_sparse_core_info()
    nc, ns = info.num_cores, info.num_subcores
    nw = nc * ns
    assert n_pairs % (nw * _SUB) == 0
    sub_per_w = n_pairs // (nw * _SUB)
    assert sub_per_w % _GROUP == 0
    n_groups = sub_per_w // _GROUP
    rows_per_group = _GROUP * _SUB
    rows_per_w = sub_per_w * _SUB

    mesh = plsc.VectorSubcoreMesh(core_axis_name="c", subcore_axis_name="s")

    @functools.partial(
        pl.kernel,
        mesh=mesh,
        compiler_params=pltpu.CompilerParams(use_tc_tiling_on_sc=False),
        out_type=jax.ShapeDtypeStruct((n_pairs, 2 * embed), jnp.float32),
        scratch_types=[
            pltpu.VMEM((sub_per_w, _SUB), jnp.int32),
            pltpu.VMEM((sub_per_w, _SUB), jnp.int32),
            pltpu.VMEM((rows_per_group, embed), jnp.float32),
            pltpu.VMEM((rows_per_group, embed), jnp.float32),
            pltpu.SemaphoreType.DMA,
        ],
    )
    def gather_kernel(table_hbm, idxe_hbm, idxo_hbm, out_hbm,
                      idxe_v, idxo_v, rows_e, rows_o, gsem):
        wid = lax.axis_index("s") * nc + lax.axis_index("c")
        pltpu.sync_copy(idxe_hbm.at[wid], idxe_v)
        pltpu.sync_copy(idxo_hbm.at[wid], idxo_v)
        for g in range(n_groups):
            copies = []
            for s in range(_GROUP):
                j = g * _GROUP + s
                dst = pl.ds(s * _SUB, _SUB)
                copies.append(pltpu.async_copy(
                    table_hbm.at[idxe_v.at[j]], rows_e.at[dst], gsem))
                copies.append(pltpu.async_copy(
                    table_hbm.at[idxo_v.at[j]], rows_o.at[dst], gsem))
            for c in copies:
                c.wait()
            row0 = wid * rows_per_w + g * rows_per_group
            pltpu.sync_copy(
                rows_e,
                out_hbm.at[pl.ds(row0, rows_per_group), pl.ds(0, embed)])
            pltpu.sync_copy(
                rows_o,
                out_hbm.at[pl.ds(row0, rows_per_group), pl.ds(embed, embed)])

    return gather_kernel(table, idx_e, idx_o)


def _mm_body(x_ref, w_ref, b_ref, o_ref):
    o_ref[...] = (
        jnp.dot(x_ref[...], w_ref[...], preferred_element_type=jnp.float32)
        + b_ref[...]
    )


def _matmul(x, wt, b2, block_m):
    m, k = x.shape
    _, n = wt.shape
    return pl.pallas_call(
        _mm_body,
        grid=(m // block_m,),
        in_specs=[
            pl.BlockSpec((block_m, k), lambda i: (i, 0)),
            pl.BlockSpec((k, n), lambda i: (0, 0)),
            pl.BlockSpec((1, n), lambda i: (0, 0)),
        ],
        out_specs=pl.BlockSpec((block_m, n), lambda i: (i, 0)),
        out_shape=jax.ShapeDtypeStruct((m, n), jnp.float32),
    )(x, wt, b2)


def kernel(inventory, node_embeds, W, b):
    batch, n = inventory.shape
    vocab, embed = node_embeds.shape
    out_dim = W.shape[0]
    n_rows = batch * n
    n_pairs = n_rows // 2
    feat = n * embed
    info = plsc.get_sparse_core_info()
    nw = info.num_cores * info.num_subcores
    idx4 = inventory.reshape(nw, n_pairs // (nw * _SUB), _SUB, 2)
    idx_e = idx4[..., 0]
    idx_o = idx4[..., 1]
    pairs = _gather_pairs(node_embeds, idx_e, idx_o, n_pairs, embed)
    x2 = pairs.reshape(batch // 2, 2 * feat)
    wt = W.T
    zeros = jnp.zeros_like(wt)
    w2 = jnp.block([[wt, zeros], [zeros, wt]])
    b2 = jnp.concatenate([b, b]).reshape(1, 2 * out_dim)
    out2 = _matmul(x2, w2, b2, block_m=512)
    return out2.reshape(batch, out_dim)


# speedup vs baseline: 2.0100x; 2.0100x over previous
import functools

import jax
import jax.numpy as jnp
from jax import lax
from jax.experimental import pallas as pl
from jax.experimental.pallas import tpu as pltpu
from jax.experimental.pallas import tpu_sc as plsc

_SUB = 128


def _gather_pairs_chunked(table, idx_c, n_pairs, embed, n_chunks):
    info = plsc.get_sparse_core_info()
    nc, ns = info.num_cores, info.num_subcores
    nw = nc * ns
    pairs_per_w = n_pairs // nw
    assert pairs_per_w == 2 * _SUB

    mesh = plsc.VectorSubcoreMesh(core_axis_name="c", subcore_axis_name="s")

    @functools.partial(
        pl.kernel,
        mesh=mesh,
        compiler_params=pltpu.CompilerParams(use_tc_tiling_on_sc=False),
        out_type=jax.ShapeDtypeStruct((n_chunks, n_pairs, 2 * embed),
                                      jnp.float32),
        scratch_types=[
            pltpu.VMEM((4 * n_chunks, _SUB), jnp.int32),
            pltpu.VMEM((2, 2, pairs_per_w, embed), jnp.float32),
            pltpu.SemaphoreType.DMA,
            pltpu.SemaphoreType.DMA,
            pltpu.SemaphoreType.DMA,
            pltpu.SemaphoreType.DMA,
        ],
    )
    def gather_kernel(table_hbm, idx_hbm, out_hbm, idx_v, bufs, g0, g1, w0, w1):
        wid = lax.axis_index("s") * nc + lax.axis_index("c")
        p0 = wid * pairs_per_w
        gsem = (g0, g1)
        wsem = (w0, w1)
        pltpu.sync_copy(idx_hbm.at[wid], idx_v)

        def fire(r):
            cur = r % 2
            return [
                pltpu.async_copy(
                    table_hbm.at[idx_v.at[4 * r + 2 * par + h]],
                    bufs.at[cur, par, pl.ds(h * _SUB, _SUB)],
                    gsem[cur],
                )
                for par in (0, 1) for h in (0, 1)
            ]

        def put(r):
            cur = r % 2
            return [
                pltpu.async_copy(
                    bufs.at[cur, par],
                    out_hbm.at[r, pl.ds(p0, pairs_per_w),
                               pl.ds(par * embed, embed)],
                    wsem[cur],
                )
                for par in (0, 1)
            ]

        pending_g = fire(0)
        pending_w = [None, None]
        for r in range(n_chunks):
            cur = r % 2
            if r + 1 < n_chunks:
                if pending_w[1 - cur] is not None:
                    for c in pending_w[1 - cur]:
                        c.wait()
                next_g = fire(r + 1)
            for c in pending_g:
                c.wait()
            if r + 1 < n_chunks:
                pending_g = next_g
            pending_w[cur] = put(r)
        for side in (0, 1):
            for c in pending_w[side]:
                c.wait()

    return gather_kernel(table, idx_c)


def _mm_body(x_ref, w_ref, b_ref, o_ref):
    n_chunks = x_ref.shape[0]
    acc = jnp.dot(x_ref[0], w_ref[0], preferred_element_type=jnp.float32)
    for r in range(1, n_chunks):
        acc += jnp.dot(x_ref[r], w_ref[r], preferred_element_type=jnp.float32)
    o_ref[...] = acc + b_ref[...]


def _matmul_chunks(x9, w9, b2, block_m):
    n_chunks, m, k = x9.shape
    n = w9.shape[2]
    return pl.pallas_call(
        _mm_body,
        grid=(m // block_m,),
        in_specs=[
            pl.BlockSpec((n_chunks, block_m, k), lambda i: (0, i, 0)),
            pl.BlockSpec((n_chunks, k, n), lambda i: (0, 0, 0)),
            pl.BlockSpec((1, n), lambda i: (0, 0)),
        ],
        out_specs=pl.BlockSpec((block_m, n), lambda i: (i, 0)),
        out_shape=jax.ShapeDtypeStruct((m, n), jnp.float32),
    )(x9, w9, b2)


def kernel(inventory, node_embeds, W, b):
    batch, n = inventory.shape
    vocab, embed = node_embeds.shape
    out_dim = W.shape[0]
    n_pairs = batch // 2
    info = plsc.get_sparse_core_info()
    nw = info.num_cores * info.num_subcores

    idx_c = (inventory.reshape(nw, 2, _SUB, 2, n)
             .transpose(0, 3, 4, 1, 2)
             .reshape(nw, 4 * n, _SUB))

    x9 = _gather_pairs_chunked(node_embeds, idx_c, n_pairs, embed, n)

    wt = W.T
    blocks = []
    for r in range(n):
        cols = []
        for u in (2 * r, 2 * r + 1):
            b01, j = divmod(u, n)
            piece = wt[j * embed:(j + 1) * embed]
            zero = jnp.zeros_like(piece)
            half = (jnp.concatenate([piece, zero], axis=1) if b01 == 0
                    else jnp.concatenate([zero, piece], axis=1))
            cols.append(half)
        blocks.append(jnp.concatenate(cols, axis=0))
    w9 = jnp.stack(blocks)
    b2 = jnp.concatenate([b, b]).reshape(1, 2 * out_dim)

    out2 = _matmul_chunks(x9, w9, b2, block_m=512)
    return out2.reshape(batch, out_dim)
